# baseline (speedup 1.0000x reference)
import functools

import jax
import jax.numpy as jnp
from jax.experimental import pallas as pl
from jax.experimental.pallas import tpu as pltpu

N = 5000
NFEAT = 256
HID = 32
NHEADS = 4
NCLASS = 10
FH = NHEADS * HID
SMOO = 0.5
ALPHA = 0.2

BM_B = 200
NB_B = N // BM_B
BM_D = 200
NB_D = N // BM_D


def _group_mats():
    lane = jax.lax.broadcasted_iota(jnp.int32, (FH, NHEADS), 0)
    col = jax.lax.broadcasted_iota(jnp.int32, (FH, NHEADS), 1)
    g = jnp.where(lane // HID == col, 1.0, 0.0).astype(jnp.float32)
    colt = jax.lax.broadcasted_iota(jnp.int32, (NHEADS, FH), 0)
    lanet = jax.lax.broadcasted_iota(jnp.int32, (NHEADS, FH), 1)
    gt = jnp.where(lanet // HID == colt, 1.0, 0.0).astype(jnp.float32)
    return g, gt


def _fused_kernel(x_ref, g0_ref, b0_ref, w_ref, aself_ref, anb_ref,
                  g1_ref, b1_ref, gcw_ref, gcb_ref,
                  at_ref, s1_ref, s2_ref, s3_ref, adj_ref,
                  out_ref, h_scr, mix_scr, sup_scr):
    i = pl.program_id(0)
    dot = functools.partial(jnp.dot, preferred_element_type=jnp.float32)

    @pl.when(i == 0)
    def _proj():
        xv = x_ref[...]
        mu = jnp.mean(xv, axis=0, keepdims=True)
        var = jnp.mean((xv - mu) ** 2, axis=0, keepdims=True)
        xb = g0_ref[...] * (xv - mu) * jax.lax.rsqrt(var + 1e-5) + b0_ref[...]
        h_scr[...] = dot(xb, w_ref[...])

    @pl.when(i < NB_B)
    def _att():
        h_blk = h_scr[pl.ds(i * BM_B, BM_B), :]
        hf = h_scr[...]
        g, gt = _group_mats()
        a_nb = anb_ref[...]

        ch0 = dot(at_ref[...], hf)
        p1 = dot(s1_ref[...], hf)
        p2 = dot(s2_ref[...], hf)
        p3 = dot(s3_ref[...], hf)
        sq1, sq2, sq3 = p1 * p1, p2 * p2, p3 * p3
        ch1, ch2, ch3 = sq1 * sq1, sq2 * sq2, sq3 * sq3

        e_self = dot(h_blk * aself_ref[...], g)

        def score(ch):
            e = e_self + dot(ch * a_nb, g)
            return jnp.where(e >= 0, e, ALPHA * e)

        e0, e1, e2, e3 = score(ch0), score(ch1), score(ch2), score(ch3)
        m = jnp.maximum(jnp.maximum(e0, e1), jnp.maximum(e2, e3))
        w0 = jnp.exp(e0 - m)
        w1 = jnp.exp(e1 - m)
        w2 = jnp.exp(e2 - m)
        w3 = jnp.exp(e3 - m)
        inv = 1.0 / (w0 + w1 + w2 + w3)
        mix_scr[pl.ds(i * BM_B, BM_B), :] = (
            dot(w0 * inv, gt) * ch0 + dot(w1 * inv, gt) * ch1
            + dot(w2 * inv, gt) * ch2 + dot(w3 * inv, gt) * ch3)

    @pl.when(i == NB_B)
    def _support():
        mx = mix_scr[...]
        mu = jnp.mean(mx, axis=0, keepdims=True)
        var = jnp.mean((mx - mu) ** 2, axis=0, keepdims=True)
        xc = g1_ref[...] * (mx - mu) * jax.lax.rsqrt(var + 1e-5) + b1_ref[...]
        xc = jnp.maximum(xc, 0.0)
        sup_scr[...] = dot(xc, gcw_ref[...]) + gcb_ref[...]

    @pl.when(i >= NB_B)
    def _smooth():
        j = i - NB_B
        sup_blk = sup_scr[pl.ds(j * BM_D, BM_D), :]
        prop = dot(adj_ref[...], sup_scr[...])
        o = (SMOO * prop + sup_blk) * (1.0 / (SMOO + 1.0))
        m = jnp.max(o, axis=1, keepdims=True)
        ex = jnp.exp(o - m)
        lse = jnp.log(jnp.sum(ex, axis=1, keepdims=True))
        out_ref[...] = o - m - lse


def kernel(x, adj_p, A_tilde, s1_sct, s2_sct, s3_sct, W_heads, a_heads,
           bn0_gamma, bn0_beta, bn1_gamma, bn1_beta, gc_W, gc_b):
    f32 = jnp.float32
    w_cat = jnp.transpose(W_heads, (1, 0, 2)).reshape(NFEAT, FH)
    a_self = a_heads[:, :HID, 0].reshape(1, FH)
    a_nb = a_heads[:, HID:, 0].reshape(1, FH)
    g0 = bn0_gamma.reshape(1, NFEAT)
    b0 = bn0_beta.reshape(1, NFEAT)
    g1 = jnp.tile(bn1_gamma, NHEADS).reshape(1, FH)
    b1 = jnp.tile(bn1_beta, NHEADS).reshape(1, FH)
    gcb = gc_b.reshape(1, NCLASS)

    full = lambda *shape: pl.BlockSpec(shape, lambda i: (0,) * len(shape))
    op_spec = pl.BlockSpec((BM_B, N), lambda i: (jnp.minimum(i, NB_B - 1), 0))
    adj_spec = pl.BlockSpec((BM_D, N),
                            lambda i: (jnp.maximum(i - NB_B, 0), 0))
    out = pl.pallas_call(
        _fused_kernel,
        grid=(NB_B + NB_D,),
        in_specs=[full(N, NFEAT), full(1, NFEAT), full(1, NFEAT),
                  full(NFEAT, FH), full(1, FH), full(1, FH),
                  full(1, FH), full(1, FH), full(FH, NCLASS),
                  full(1, NCLASS),
                  op_spec, op_spec, op_spec, op_spec, adj_spec],
        out_specs=pl.BlockSpec((BM_D, NCLASS),
                               lambda i: (jnp.maximum(i - NB_B, 0), 0)),
        out_shape=jax.ShapeDtypeStruct((N, NCLASS), f32),
        scratch_shapes=[pltpu.VMEM((N, FH), f32), pltpu.VMEM((N, FH), f32),
                        pltpu.VMEM((N, NCLASS), f32)],
    )(x, g0, b0, w_cat, a_self, a_nb, g1, b1, gc_W, gcb,
      A_tilde, s1_sct, s2_sct, s3_sct, adj_p)
    return out


# speedup vs baseline: 1.0269x; 1.0269x over previous
import functools

import jax
import jax.numpy as jnp
from jax.experimental import pallas as pl
from jax.experimental.pallas import tpu as pltpu

N = 5000
NFEAT = 256
HID = 32
NHEADS = 4
NCLASS = 10
FH = NHEADS * HID
SMOO = 0.5
ALPHA = 0.2

BM_B = 200
BM_D = 1000


def _group_mats():
    lane = jax.lax.broadcasted_iota(jnp.int32, (FH, NHEADS), 0)
    col = jax.lax.broadcasted_iota(jnp.int32, (FH, NHEADS), 1)
    g = jnp.where(lane // HID == col, 1.0, 0.0).astype(jnp.float32)
    colt = jax.lax.broadcasted_iota(jnp.int32, (NHEADS, FH), 0)
    lanet = jax.lax.broadcasted_iota(jnp.int32, (NHEADS, FH), 1)
    gt = jnp.where(lanet // HID == colt, 1.0, 0.0).astype(jnp.float32)
    return g, gt


def _att_kernel(x_ref, g0_ref, b0_ref, w_ref, aself_ref, anb_ref,
                at_ref, s1_ref, s2_ref, s3_ref, mix_ref, h_scr):
    i = pl.program_id(0)
    dot = functools.partial(jnp.dot, preferred_element_type=jnp.float32)

    @pl.when(i == 0)
    def _proj():
        xv = x_ref[...]
        mu = jnp.mean(xv, axis=0, keepdims=True)
        var = jnp.mean((xv - mu) ** 2, axis=0, keepdims=True)
        xb = g0_ref[...] * (xv - mu) * jax.lax.rsqrt(var + 1e-5) + b0_ref[...]
        h_scr[...] = dot(xb, w_ref[...])

    @pl.when(i > 0)
    def _att():
        j = i - 1
        h_blk = h_scr[pl.ds(j * BM_B, BM_B), :]
        hf = h_scr[...]
        g, gt = _group_mats()
        a_nb = anb_ref[...]

        ch0 = dot(at_ref[...], hf)
        p1 = dot(s1_ref[...], hf)
        p2 = dot(s2_ref[...], hf)
        p3 = dot(s3_ref[...], hf)
        sq1, sq2, sq3 = p1 * p1, p2 * p2, p3 * p3
        ch1, ch2, ch3 = sq1 * sq1, sq2 * sq2, sq3 * sq3

        e_self = dot(h_blk * aself_ref[...], g)

        def score(ch):
            e = e_self + dot(ch * a_nb, g)
            return jnp.where(e >= 0, e, ALPHA * e)

        e0, e1, e2, e3 = score(ch0), score(ch1), score(ch2), score(ch3)
        m = jnp.maximum(jnp.maximum(e0, e1), jnp.maximum(e2, e3))
        w0 = jnp.exp(e0 - m)
        w1 = jnp.exp(e1 - m)
        w2 = jnp.exp(e2 - m)
        w3 = jnp.exp(e3 - m)
        inv = 1.0 / (w0 + w1 + w2 + w3)
        mix_ref[...] = (dot(w0 * inv, gt) * ch0 + dot(w1 * inv, gt) * ch1
                        + dot(w2 * inv, gt) * ch2 + dot(w3 * inv, gt) * ch3)


def _smooth_kernel(mix_ref, g1_ref, b1_ref, w_ref, bias_ref, adj_ref,
                   out_ref, sup_scr):
    i = pl.program_id(0)

    @pl.when(i == 0)
    def _support():
        mx = mix_ref[...]
        mu = jnp.mean(mx, axis=0, keepdims=True)
        var = jnp.mean((mx - mu) ** 2, axis=0, keepdims=True)
        xc = g1_ref[...] * (mx - mu) * jax.lax.rsqrt(var + 1e-5) + b1_ref[...]
        xc = jnp.maximum(xc, 0.0)
        sup_scr[...] = (jnp.dot(xc, w_ref[...],
                                preferred_element_type=jnp.float32)
                        + bias_ref[...])

    @pl.when(i > 0)
    def _smooth():
        j = i - 1
        sup_blk = sup_scr[pl.ds(j * BM_D, BM_D), :]
        prop = jnp.dot(adj_ref[...], sup_scr[...],
                       preferred_element_type=jnp.float32)
        o = (SMOO * prop + sup_blk) * (1.0 / (SMOO + 1.0))
        m = jnp.max(o, axis=1, keepdims=True)
        ex = jnp.exp(o - m)
        lse = jnp.log(jnp.sum(ex, axis=1, keepdims=True))
        out_ref[...] = o - m - lse


def kernel(x, adj_p, A_tilde, s1_sct, s2_sct, s3_sct, W_heads, a_heads,
           bn0_gamma, bn0_beta, bn1_gamma, bn1_beta, gc_W, gc_b):
    f32 = jnp.float32
    w_cat = jnp.transpose(W_heads, (1, 0, 2)).reshape(NFEAT, FH)
    a_self = a_heads[:, :HID, 0].reshape(1, FH)
    a_nb = a_heads[:, HID:, 0].reshape(1, FH)
    g0 = bn0_gamma.reshape(1, NFEAT)
    b0 = bn0_beta.reshape(1, NFEAT)
    g1 = jnp.tile(bn1_gamma, NHEADS).reshape(1, FH)
    b1 = jnp.tile(bn1_beta, NHEADS).reshape(1, FH)
    gcb = gc_b.reshape(1, NCLASS)

    full = lambda *shape: pl.BlockSpec(shape, lambda i: (0,) * len(shape))
    lag = lambda i: (jnp.maximum(i - 1, 0), 0)
    op_spec = pl.BlockSpec((BM_B, N), lag)
    mixed = pl.pallas_call(
        _att_kernel,
        grid=(N // BM_B + 1,),
        in_specs=[full(N, NFEAT), full(1, NFEAT), full(1, NFEAT),
                  full(NFEAT, FH), full(1, FH), full(1, FH),
                  op_spec, op_spec, op_spec, op_spec],
        out_specs=pl.BlockSpec((BM_B, FH), lag),
        out_shape=jax.ShapeDtypeStruct((N, FH), f32),
        scratch_shapes=[pltpu.VMEM((N, FH), f32)],
    )(x, g0, b0, w_cat, a_self, a_nb, A_tilde, s1_sct, s2_sct, s3_sct)

    out = pl.pallas_call(
        _smooth_kernel,
        grid=(N // BM_D + 1,),
        in_specs=[full(N, FH), full(1, FH), full(1, FH),
                  full(FH, NCLASS), full(1, NCLASS),
                  pl.BlockSpec((BM_D, N), lag)],
        out_specs=pl.BlockSpec((BM_D, NCLASS), lag),
        out_shape=jax.ShapeDtypeStruct((N, NCLASS), f32),
        scratch_shapes=[pltpu.VMEM((N, NCLASS), f32)],
    )(mixed, g1, b1, gc_W, gcb, adj_p)
    return out


# speedup vs baseline: 1.0418x; 1.0145x over previous
import functools

import jax
import jax.numpy as jnp
from jax.experimental import pallas as pl
from jax.experimental.pallas import tpu as pltpu

N = 5000
NFEAT = 256
HID = 32
NHEADS = 4
NCLASS = 10
FH = NHEADS * HID
SMOO = 0.5
ALPHA = 0.2

BM_B = 200
BM_D = 1000


def _group_mats():
    lane = jax.lax.broadcasted_iota(jnp.int32, (FH, NHEADS), 0)
    col = jax.lax.broadcasted_iota(jnp.int32, (FH, NHEADS), 1)
    g = jnp.where(lane // HID == col, 1.0, 0.0).astype(jnp.float32)
    colt = jax.lax.broadcasted_iota(jnp.int32, (NHEADS, FH), 0)
    lanet = jax.lax.broadcasted_iota(jnp.int32, (NHEADS, FH), 1)
    gt = jnp.where(lanet // HID == colt, 1.0, 0.0).astype(jnp.float32)
    return g, gt


def _att_kernel(x_ref, g0_ref, b0_ref, w_ref, aself_ref, anb_ref,
                at_ref, s1_ref, s2_ref, s3_ref, mix_ref, h_scr):
    i = pl.program_id(0)
    dot = functools.partial(jnp.dot, preferred_element_type=jnp.float32)

    @pl.when(i == 0)
    def _proj():
        xv = x_ref[...]
        mu = jnp.mean(xv, axis=0, keepdims=True)
        var = jnp.mean((xv - mu) ** 2, axis=0, keepdims=True)
        xb = g0_ref[...] * (xv - mu) * jax.lax.rsqrt(var + 1e-5) + b0_ref[...]
        h_scr[...] = dot(xb, w_ref[...])

    h_blk = h_scr[pl.ds(i * BM_B, BM_B), :]
    hf = h_scr[...]
    g, gt = _group_mats()
    a_nb = anb_ref[...]

    ch0 = dot(at_ref[...], hf)
    p1 = dot(s1_ref[...], hf)
    p2 = dot(s2_ref[...], hf)
    p3 = dot(s3_ref[...], hf)
    sq1, sq2, sq3 = p1 * p1, p2 * p2, p3 * p3
    ch1, ch2, ch3 = sq1 * sq1, sq2 * sq2, sq3 * sq3

    e_self = dot(h_blk * aself_ref[...], g)

    def score(ch):
        e = e_self + dot(ch * a_nb, g)
        return jnp.where(e >= 0, e, ALPHA * e)

    e0, e1, e2, e3 = score(ch0), score(ch1), score(ch2), score(ch3)
    m = jnp.maximum(jnp.maximum(e0, e1), jnp.maximum(e2, e3))
    w0 = jnp.exp(e0 - m)
    w1 = jnp.exp(e1 - m)
    w2 = jnp.exp(e2 - m)
    w3 = jnp.exp(e3 - m)
    inv = 1.0 / (w0 + w1 + w2 + w3)
    mix_ref[...] = (dot(w0 * inv, gt) * ch0 + dot(w1 * inv, gt) * ch1
                    + dot(w2 * inv, gt) * ch2 + dot(w3 * inv, gt) * ch3)


def _smooth_kernel(mix_ref, g1_ref, b1_ref, w_ref, bias_ref, adj_ref,
                   out_ref, sup_scr):
    i = pl.program_id(0)

    @pl.when(i == 0)
    def _support():
        mx = mix_ref[...]
        mu = jnp.mean(mx, axis=0, keepdims=True)
        var = jnp.mean((mx - mu) ** 2, axis=0, keepdims=True)
        xc = g1_ref[...] * (mx - mu) * jax.lax.rsqrt(var + 1e-5) + b1_ref[...]
        xc = jnp.maximum(xc, 0.0)
        sup_scr[...] = (jnp.dot(xc, w_ref[...],
                                preferred_element_type=jnp.float32)
                        + bias_ref[...])

    sup_blk = sup_scr[pl.ds(i * BM_D, BM_D), :]
    prop = jnp.dot(adj_ref[...], sup_scr[...],
                   preferred_element_type=jnp.float32)
    o = (SMOO * prop + sup_blk) * (1.0 / (SMOO + 1.0))
    m = jnp.max(o, axis=1, keepdims=True)
    ex = jnp.exp(o - m)
    lse = jnp.log(jnp.sum(ex, axis=1, keepdims=True))
    out_ref[...] = o - m - lse


def kernel(x, adj_p, A_tilde, s1_sct, s2_sct, s3_sct, W_heads, a_heads,
           bn0_gamma, bn0_beta, bn1_gamma, bn1_beta, gc_W, gc_b):
    f32 = jnp.float32
    w_cat = jnp.transpose(W_heads, (1, 0, 2)).reshape(NFEAT, FH)
    a_self = a_heads[:, :HID, 0].reshape(1, FH)
    a_nb = a_heads[:, HID:, 0].reshape(1, FH)
    g0 = bn0_gamma.reshape(1, NFEAT)
    b0 = bn0_beta.reshape(1, NFEAT)
    g1 = jnp.tile(bn1_gamma, NHEADS).reshape(1, FH)
    b1 = jnp.tile(bn1_beta, NHEADS).reshape(1, FH)
    gcb = gc_b.reshape(1, NCLASS)

    full = lambda *shape: pl.BlockSpec(shape, lambda i: (0,) * len(shape))
    op_spec = pl.BlockSpec((BM_B, N), lambda i: (i, 0))
    mixed = pl.pallas_call(
        _att_kernel,
        grid=(N // BM_B,),
        in_specs=[full(N, NFEAT), full(1, NFEAT), full(1, NFEAT),
                  full(NFEAT, FH), full(1, FH), full(1, FH),
                  op_spec, op_spec, op_spec, op_spec],
        out_specs=pl.BlockSpec((BM_B, FH), lambda i: (i, 0)),
        out_shape=jax.ShapeDtypeStruct((N, FH), f32),
        scratch_shapes=[pltpu.VMEM((N, FH), f32)],
    )(x, g0, b0, w_cat, a_self, a_nb, A_tilde, s1_sct, s2_sct, s3_sct)

    out = pl.pallas_call(
        _smooth_kernel,
        grid=(N // BM_D,),
        in_specs=[full(N, FH), full(1, FH), full(1, FH),
                  full(FH, NCLASS), full(1, NCLASS),
                  pl.BlockSpec((BM_D, N), lambda i: (i, 0))],
        out_specs=pl.BlockSpec((BM_D, NCLASS), lambda i: (i, 0)),
        out_shape=jax.ShapeDtypeStruct((N, NCLASS), f32),
        scratch_shapes=[pltpu.VMEM((N, NCLASS), f32)],
    )(mixed, g1, b1, gc_W, gcb, adj_p)
    return out
